# initial kernel scaffold (unmeasured)
import jax
import jax.numpy as jnp
from jax import lax
from jax.experimental import pallas as pl
from jax.experimental.pallas import tpu as pltpu


def kernel(
    x,
):
    def body(*refs):
        pass

    out_shape = jax.ShapeDtypeStruct(..., jnp.float32)
    return pl.pallas_call(body, out_shape=out_shape)(...)


# baseline (device time: 215744 ns/iter reference)
import jax
import jax.numpy as jnp
from jax import lax
from jax.experimental import pallas as pl
from jax.experimental.pallas import tpu as pltpu


def kernel(x):
    x16 = x.astype(jnp.bfloat16)
    m, n = x16.shape
    half = m // 2

    def body(x_ref, out_ref, recv_ref, s_sem1, r_sem1, s_sem2, r_sem2):
        my_x = lax.axis_index("x")
        my_y = lax.axis_index("y")
        y_nbr = (my_x, 1 - my_y)
        x_nbr = (1 - my_x, my_y)

        barrier = pltpu.get_barrier_semaphore()
        for nbr in (y_nbr, x_nbr):
            pl.semaphore_signal(
                barrier, inc=1, device_id=nbr,
                device_id_type=pl.DeviceIdType.MESH,
            )
        pl.semaphore_wait(barrier, 2)

        off = my_x * half

        rdma1 = pltpu.make_async_remote_copy(
            src_ref=x_ref.at[pl.ds(off, half)],
            dst_ref=recv_ref,
            send_sem=s_sem1,
            recv_sem=r_sem1,
            device_id=y_nbr,
            device_id_type=pl.DeviceIdType.MESH,
        )
        rdma1.start()
        rdma1.wait()
        out_ref[pl.ds(off, half), :] = x_ref[pl.ds(off, half), :] + recv_ref[:, :]

        rdma2 = pltpu.make_async_remote_copy(
            src_ref=out_ref.at[pl.ds(off, half)],
            dst_ref=out_ref.at[pl.ds(off, half)],
            send_sem=s_sem2,
            recv_sem=r_sem2,
            device_id=x_nbr,
            device_id_type=pl.DeviceIdType.MESH,
        )
        rdma2.start()
        rdma2.wait()

    return pl.pallas_call(
        body,
        out_shape=jax.ShapeDtypeStruct((m, n), jnp.bfloat16),
        in_specs=[pl.BlockSpec(memory_space=pltpu.VMEM)],
        out_specs=pl.BlockSpec(memory_space=pltpu.VMEM),
        scratch_shapes=[
            pltpu.VMEM((half, n), jnp.bfloat16),
            pltpu.SemaphoreType.DMA,
            pltpu.SemaphoreType.DMA,
            pltpu.SemaphoreType.DMA,
            pltpu.SemaphoreType.DMA,
        ],
        compiler_params=pltpu.CompilerParams(collective_id=0),
    )(x16)


# device time: 131116 ns/iter; 1.6454x vs baseline; 1.6454x over previous
import jax
import jax.numpy as jnp
from jax import lax
from jax.experimental import pallas as pl
from jax.experimental.pallas import tpu as pltpu

N_CHUNKS = 16


def kernel(x):
    x16 = x.astype(jnp.bfloat16)
    m, n = x16.shape
    half = m // 2
    rows = half // N_CHUNKS

    def body(x_ref, out_ref, recv_ref, s_sem1, r_sem1, s_sem2, r_sem2):
        my_x = lax.axis_index("x")
        my_y = lax.axis_index("y")
        y_nbr = (my_x, 1 - my_y)
        x_nbr = (1 - my_x, my_y)

        barrier = pltpu.get_barrier_semaphore()
        for nbr in (y_nbr, x_nbr):
            pl.semaphore_signal(
                barrier, inc=1, device_id=nbr,
                device_id_type=pl.DeviceIdType.MESH,
            )
        pl.semaphore_wait(barrier, 2)

        off = my_x * half

        phase1 = []
        for c in range(N_CHUNKS):
            rdma = pltpu.make_async_remote_copy(
                src_ref=x_ref.at[pl.ds(off + c * rows, rows)],
                dst_ref=recv_ref.at[pl.ds(c * rows, rows)],
                send_sem=s_sem1.at[c],
                recv_sem=r_sem1.at[c],
                device_id=y_nbr,
                device_id_type=pl.DeviceIdType.MESH,
            )
            rdma.start()
            phase1.append(rdma)

        phase2 = []
        for c in range(N_CHUNKS):
            phase1[c].wait_recv()
            sl = pl.ds(off + c * rows, rows)
            out_ref[sl, :] = x_ref[sl, :] + recv_ref[pl.ds(c * rows, rows), :]
            rdma = pltpu.make_async_remote_copy(
                src_ref=out_ref.at[sl],
                dst_ref=out_ref.at[sl],
                send_sem=s_sem2.at[c],
                recv_sem=r_sem2.at[c],
                device_id=x_nbr,
                device_id_type=pl.DeviceIdType.MESH,
            )
            rdma.start()
            phase2.append(rdma)

        for c in range(N_CHUNKS):
            phase1[c].wait_send()
            phase2[c].wait_send()
            phase2[c].wait_recv()

    return pl.pallas_call(
        body,
        out_shape=jax.ShapeDtypeStruct((m, n), jnp.bfloat16),
        in_specs=[pl.BlockSpec(memory_space=pltpu.VMEM)],
        out_specs=pl.BlockSpec(memory_space=pltpu.VMEM),
        scratch_shapes=[
            pltpu.VMEM((half, n), jnp.bfloat16),
            pltpu.SemaphoreType.DMA((N_CHUNKS,)),
            pltpu.SemaphoreType.DMA((N_CHUNKS,)),
            pltpu.SemaphoreType.DMA((N_CHUNKS,)),
            pltpu.SemaphoreType.DMA((N_CHUNKS,)),
        ],
        compiler_params=pltpu.CompilerParams(collective_id=0),
    )(x16)


# device time: 113312 ns/iter; 1.9040x vs baseline; 1.1571x over previous
import jax
import jax.numpy as jnp
from jax import lax
from jax.experimental import pallas as pl
from jax.experimental.pallas import tpu as pltpu

N_CHUNKS = 16


def kernel(x):
    m, n = x.shape
    half = m // 2
    rows = half // N_CHUNKS

    def body(x_hbm, out_ref, xv_ref, send_ref, recv_ref, load_sem,
             s_sem1, r_sem1, s_sem2, r_sem2):
        my_x = lax.axis_index("x")
        my_y = lax.axis_index("y")
        y_nbr = (my_x, 1 - my_y)
        x_nbr = (1 - my_x, my_y)

        off = my_x * half

        load = pltpu.make_async_copy(
            x_hbm.at[pl.ds(off, half)], xv_ref, load_sem
        )
        load.start()

        barrier = pltpu.get_barrier_semaphore()
        for nbr in (y_nbr, x_nbr):
            pl.semaphore_signal(
                barrier, inc=1, device_id=nbr,
                device_id_type=pl.DeviceIdType.MESH,
            )
        pl.semaphore_wait(barrier, 2)
        load.wait()

        phase1 = []
        for c in range(N_CHUNKS):
            csl = pl.ds(c * rows, rows)
            send_ref[csl, :] = xv_ref[csl, :].astype(jnp.bfloat16)
            rdma = pltpu.make_async_remote_copy(
                src_ref=send_ref.at[csl],
                dst_ref=recv_ref.at[csl],
                send_sem=s_sem1.at[c],
                recv_sem=r_sem1.at[c],
                device_id=y_nbr,
                device_id_type=pl.DeviceIdType.MESH,
            )
            rdma.start()
            phase1.append(rdma)

        phase2 = []
        for c in range(N_CHUNKS):
            phase1[c].wait_recv()
            csl = pl.ds(c * rows, rows)
            sl = pl.ds(off + c * rows, rows)
            out_ref[sl, :] = send_ref[csl, :] + recv_ref[csl, :]
            rdma = pltpu.make_async_remote_copy(
                src_ref=out_ref.at[sl],
                dst_ref=out_ref.at[sl],
                send_sem=s_sem2.at[c],
                recv_sem=r_sem2.at[c],
                device_id=x_nbr,
                device_id_type=pl.DeviceIdType.MESH,
            )
            rdma.start()
            phase2.append(rdma)

        for c in range(N_CHUNKS):
            phase1[c].wait_send()
            phase2[c].wait_send()
            phase2[c].wait_recv()

    return pl.pallas_call(
        body,
        out_shape=jax.ShapeDtypeStruct((m, n), jnp.bfloat16),
        in_specs=[pl.BlockSpec(memory_space=pltpu.MemorySpace.HBM)],
        out_specs=pl.BlockSpec(memory_space=pltpu.VMEM),
        scratch_shapes=[
            pltpu.VMEM((half, n), jnp.float32),
            pltpu.VMEM((half, n), jnp.bfloat16),
            pltpu.VMEM((half, n), jnp.bfloat16),
            pltpu.SemaphoreType.DMA,
            pltpu.SemaphoreType.DMA((N_CHUNKS,)),
            pltpu.SemaphoreType.DMA((N_CHUNKS,)),
            pltpu.SemaphoreType.DMA((N_CHUNKS,)),
            pltpu.SemaphoreType.DMA((N_CHUNKS,)),
        ],
        compiler_params=pltpu.CompilerParams(collective_id=0),
    )(x)
